# Initial kernel scaffold; baseline (speedup 1.0000x reference)
import jax
import jax.numpy as jnp
from jax.experimental import pallas as pl


def kernel(x, edge_index, Wl1, bl1, Wr1, Wl2, bl2, Wr2, Wh, bh):
    raise NotImplementedError("write your pallas kernel here")


# speedup vs baseline: 8.7031x; 8.7031x over previous
import functools

import jax
import jax.numpy as jnp
from jax import lax
from jax.experimental import pallas as pl
from jax.experimental.pallas import tpu as pltpu
from jax.experimental.pallas import tpu_sc as plsc

N_NODES = 10000
N_EDGES = 320000
IN_CH = 128
HID = 32
OUT_DIM = HID // 2
CW = 16

NW = 32
CH = 128
K = 8
RPW = 80
G = RPW // K
EROWS = NW * RPW
EPAD = EROWS * CH
NPAD = 10112
RPS = NPAD // 16


def _edge_pass_body(with_counts, *refs):
    if with_counts:
        (table, srcm, dstm, z32, z16, ones_h,
         acc_out, cnt_out, acc_sh, cnt_sh, ones_v,
         src_v, dst_v, rows_v, sem) = refs
    else:
        (table, srcm, dstm, z32,
         acc_out, acc_sh, src_v, dst_v, rows_v, sem) = refs

    c = lax.axis_index("c")
    s = lax.axis_index("s")
    wid = s * 2 + c

    sl = pl.ds(s * RPS, RPS)
    pltpu.sync_copy(z32.at[sl], acc_sh.at[sl])
    if with_counts:
        pltpu.sync_copy(z16.at[sl], cnt_sh.at[sl])
        pltpu.sync_copy(ones_h, ones_v)
    plsc.subcore_barrier()

    def super_chunk(g, carry):
        r0 = wid * RPW + g * K
        pltpu.sync_copy(srcm.at[pl.ds(r0, K)], src_v)
        pltpu.sync_copy(dstm.at[pl.ds(r0, K)], dst_v)
        descs = [
            pltpu.async_copy(table.at[src_v.at[j]], rows_v.at[j], sem)
            for j in range(K)
        ]
        for d in descs:
            d.wait()
        for j in range(K):
            pltpu.sync_copy(rows_v.at[j], acc_sh.at[dst_v.at[j]], add=True)
            if with_counts:
                pltpu.sync_copy(ones_v, cnt_sh.at[dst_v.at[j]], add=True)
        return carry

    lax.fori_loop(0, G, super_chunk, 0)
    plsc.subcore_barrier()

    pltpu.sync_copy(acc_sh.at[sl], acc_out.at[c, sl])
    if with_counts:
        pltpu.sync_copy(cnt_sh.at[sl], cnt_out.at[c, sl])


@functools.lru_cache(maxsize=None)
def _make_edge_pass(with_counts):
    f32, i32 = jnp.float32, jnp.int32
    outs = [jax.ShapeDtypeStruct((2, NPAD, HID), f32)]
    scratch = [pltpu.VMEM_SHARED((NPAD, HID), f32)]
    if with_counts:
        outs.append(jax.ShapeDtypeStruct((2, NPAD, CW), f32))
        scratch += [pltpu.VMEM_SHARED((NPAD, CW), f32), pltpu.VMEM((CH, CW), f32)]
    scratch += [
        pltpu.VMEM((K, CH), i32),
        pltpu.VMEM((K, CH), i32),
        pltpu.VMEM((K, CH, HID), f32),
        pltpu.SemaphoreType.DMA,
    ]
    mesh = plsc.VectorSubcoreMesh(core_axis_name="c", subcore_axis_name="s")
    return pl.kernel(
        functools.partial(_edge_pass_body, with_counts),
        out_type=tuple(outs),
        mesh=mesh,
        scratch_types=scratch,
        compiler_params=pltpu.CompilerParams(use_tc_tiling_on_sc=False),
        name=f"sage_edge_pass_{'cnt' if with_counts else 'nocnt'}",
    )


def _dot_t(a, w):
    return lax.dot_general(a, w, (((1,), (1,)), ((), ())),
                           preferred_element_type=jnp.float32)


def _pre_body(x, wl, wr, bl, p_out, r_out):
    xv = x[...]
    p_out[...] = _dot_t(xv, wl[...])
    r_out[...] = _dot_t(xv, wr[...]) + bl[...]


def _mid_body(accA, accB, cntA, cntB, rpb, wl2, wr2, bl2, p_out, r_out, inv_out):
    agg = accA[...][:N_NODES] + accB[...][:N_NODES]
    cnt = cntA[...][:N_NODES, 0:1] + cntB[...][:N_NODES, 0:1]
    inv = 1.0 / jnp.maximum(cnt, 1.0)
    h1 = jnp.maximum(agg * inv + rpb[...], 0.0)
    p_out[...] = _dot_t(h1, wl2[...])
    r_out[...] = _dot_t(h1, wr2[...]) + bl2[...]
    inv_out[...] = inv


def _post_body(accA, accB, inv, rpb, wh, bh, z_out):
    agg = accA[...][:N_NODES] + accB[...][:N_NODES]
    h2 = jnp.maximum(agg * inv[...] + rpb[...], 0.0)
    z_out[...] = _dot_t(h2, wh[...]) + bh[...]


_f32 = jnp.float32

_pre = pl.pallas_call(
    _pre_body,
    out_shape=(jax.ShapeDtypeStruct((N_NODES, HID), _f32),
               jax.ShapeDtypeStruct((N_NODES, HID), _f32)),
)

_mid = pl.pallas_call(
    _mid_body,
    out_shape=(jax.ShapeDtypeStruct((N_NODES, HID), _f32),
               jax.ShapeDtypeStruct((N_NODES, HID), _f32),
               jax.ShapeDtypeStruct((N_NODES, 1), _f32)),
)

_post = pl.pallas_call(
    _post_body,
    out_shape=jax.ShapeDtypeStruct((N_NODES, OUT_DIM), _f32),
)


def kernel(x, edge_index, Wl1, bl1, Wr1, Wl2, bl2, Wr2, Wh, bh):
    src = edge_index[0].astype(jnp.int32)
    dst = edge_index[1].astype(jnp.int32)
    npad = EPAD - N_EDGES
    srcm = jnp.concatenate(
        [src, jnp.zeros((npad,), jnp.int32)]).reshape(EROWS, CH)
    dstm = jnp.concatenate(
        [dst, jnp.full((npad,), NPAD - 8, jnp.int32)]).reshape(EROWS, CH)
    z32 = jnp.zeros((NPAD, HID), _f32)
    z16 = jnp.zeros((NPAD, CW), _f32)
    ones_h = jnp.ones((CH, CW), _f32)

    p1, r1 = _pre(x, Wl1, Wr1, bl1.reshape(1, HID))
    acc1, cnt1 = _make_edge_pass(True)(p1, srcm, dstm, z32, z16, ones_h)
    p2, r2, inv = _mid(acc1[0], acc1[1], cnt1[0], cnt1[1], r1,
                       Wl2, Wr2, bl2.reshape(1, HID))
    (acc2,) = _make_edge_pass(False)(p2, srcm, dstm, z32)
    z = _post(acc2[0], acc2[1], inv, r2, Wh, bh.reshape(1, OUT_DIM))
    return z


# speedup vs baseline: 9.9334x; 1.1414x over previous
import functools

import jax
import jax.numpy as jnp
from jax import lax
from jax.experimental import pallas as pl
from jax.experimental.pallas import tpu as pltpu
from jax.experimental.pallas import tpu_sc as plsc

N_NODES = 10000
N_EDGES = 320000
IN_CH = 128
HID = 32
OUT_DIM = HID // 2
CW = 16

NW = 32
CH = 128
K = 8
RPW = 80
G = RPW // K
EROWS = NW * RPW
EPAD = EROWS * CH
NPAD = 10112
RPS = NPAD // 16


def _edge_pass_body(with_counts, *refs):
    if with_counts:
        (table, srcm, dstm, z32, z16, ones_h,
         acc_out, cnt_out, acc_sh, cnt_sh, ones_v,
         src_v, dst_v, rows_v, i_sem, g_sem, s_sem_a, s_sem_b) = refs
    else:
        (table, srcm, dstm, z32,
         acc_out, acc_sh,
         src_v, dst_v, rows_v, i_sem, g_sem, s_sem_a, s_sem_b) = refs

    c = lax.axis_index("c")
    s = lax.axis_index("s")
    wid = s * 2 + c

    sl = pl.ds(s * RPS, RPS)
    pltpu.sync_copy(z32.at[sl], acc_sh.at[sl])
    if with_counts:
        pltpu.sync_copy(z16.at[sl], cnt_sh.at[sl])
        pltpu.sync_copy(ones_h, ones_v)
    plsc.subcore_barrier()

    def idx_fetch(g, p):
        r0 = wid * RPW + g * K
        pltpu.async_copy(srcm.at[pl.ds(r0, K)], src_v.at[p], i_sem)
        pltpu.async_copy(dstm.at[pl.ds(r0, K)], dst_v.at[p], i_sem)

    def idx_wait(p):
        pltpu.make_async_copy(srcm.at[pl.ds(0, K)], src_v.at[p], i_sem).wait()
        pltpu.make_async_copy(dstm.at[pl.ds(0, K)], dst_v.at[p], i_sem).wait()

    def scatter_sem(p):
        return s_sem_a if p == 0 else s_sem_b

    def drain_scatters(p):
        sem = scatter_sem(p)
        for j in range(K):
            pltpu.make_async_copy(
                rows_v.at[0, j], acc_sh.at[pl.ds(0, CH)], sem).wait()
            if with_counts:
                pltpu.make_async_copy(
                    ones_v, cnt_sh.at[pl.ds(0, CH)], sem).wait()

    def run_chunk(g, p, drain_pred, prefetch_pred):
        idx_wait(p)
        gd = [
            pltpu.async_copy(table.at[src_v.at[p, j]], rows_v.at[p, j], g_sem)
            for j in range(K)
        ]

        if drain_pred is True:
            drain_scatters(1 - p)
        else:
            @pl.when(drain_pred)
            def _():
                drain_scatters(1 - p)

        @pl.when(prefetch_pred)
        def _():
            idx_fetch(g + 1, 1 - p)

        for d in gd:
            d.wait()
        sem = scatter_sem(p)
        for j in range(K):
            pltpu.async_copy(
                rows_v.at[p, j], acc_sh.at[dst_v.at[p, j]], sem, add=True)
            if with_counts:
                pltpu.async_copy(
                    ones_v, cnt_sh.at[dst_v.at[p, j]], sem, add=True)

    idx_fetch(0, 0)

    def super_chunk_pair(h, carry):
        g0 = 2 * h
        run_chunk(g0, 0, h >= 1, g0 + 1 < G)
        run_chunk(g0 + 1, 1, True, g0 + 2 < G)
        return carry

    lax.fori_loop(0, G // 2, super_chunk_pair, 0)
    drain_scatters(1)
    plsc.subcore_barrier()

    pltpu.sync_copy(acc_sh.at[sl], acc_out.at[c, sl])
    if with_counts:
        pltpu.sync_copy(cnt_sh.at[sl], cnt_out.at[c, sl])


@functools.lru_cache(maxsize=None)
def _make_edge_pass(with_counts):
    f32, i32 = jnp.float32, jnp.int32
    outs = [jax.ShapeDtypeStruct((2, NPAD, HID), f32)]
    scratch = [pltpu.VMEM_SHARED((NPAD, HID), f32)]
    if with_counts:
        outs.append(jax.ShapeDtypeStruct((2, NPAD, CW), f32))
        scratch += [pltpu.VMEM_SHARED((NPAD, CW), f32), pltpu.VMEM((CH, CW), f32)]
    scratch += [
        pltpu.VMEM((2, K, CH), i32),
        pltpu.VMEM((2, K, CH), i32),
        pltpu.VMEM((2, K, CH, HID), f32),
        pltpu.SemaphoreType.DMA,
        pltpu.SemaphoreType.DMA,
        pltpu.SemaphoreType.DMA,
        pltpu.SemaphoreType.DMA,
    ]
    mesh = plsc.VectorSubcoreMesh(core_axis_name="c", subcore_axis_name="s")
    return pl.kernel(
        functools.partial(_edge_pass_body, with_counts),
        out_type=tuple(outs),
        mesh=mesh,
        scratch_types=scratch,
        compiler_params=pltpu.CompilerParams(use_tc_tiling_on_sc=False),
        name=f"sage_edge_pass_{'cnt' if with_counts else 'nocnt'}",
    )


def _dot_t(a, w):
    return lax.dot_general(a, w, (((1,), (1,)), ((), ())),
                           preferred_element_type=jnp.float32)


def _pre_body(x, wl, wr, bl, p_out, r_out):
    xv = x[...]
    p_out[...] = _dot_t(xv, wl[...])
    r_out[...] = _dot_t(xv, wr[...]) + bl[...]


def _mid_body(accA, accB, cntA, cntB, rpb, wl2, wr2, bl2, p_out, r_out, inv_out):
    agg = accA[...][:N_NODES] + accB[...][:N_NODES]
    cnt = cntA[...][:N_NODES, 0:1] + cntB[...][:N_NODES, 0:1]
    inv = 1.0 / jnp.maximum(cnt, 1.0)
    h1 = jnp.maximum(agg * inv + rpb[...], 0.0)
    p_out[...] = _dot_t(h1, wl2[...])
    r_out[...] = _dot_t(h1, wr2[...]) + bl2[...]
    inv_out[...] = inv


def _post_body(accA, accB, inv, rpb, wh, bh, z_out):
    agg = accA[...][:N_NODES] + accB[...][:N_NODES]
    h2 = jnp.maximum(agg * inv[...] + rpb[...], 0.0)
    z_out[...] = _dot_t(h2, wh[...]) + bh[...]


_f32 = jnp.float32

_pre = pl.pallas_call(
    _pre_body,
    out_shape=(jax.ShapeDtypeStruct((N_NODES, HID), _f32),
               jax.ShapeDtypeStruct((N_NODES, HID), _f32)),
)

_mid = pl.pallas_call(
    _mid_body,
    out_shape=(jax.ShapeDtypeStruct((N_NODES, HID), _f32),
               jax.ShapeDtypeStruct((N_NODES, HID), _f32),
               jax.ShapeDtypeStruct((N_NODES, 1), _f32)),
)

_post = pl.pallas_call(
    _post_body,
    out_shape=jax.ShapeDtypeStruct((N_NODES, OUT_DIM), _f32),
)


def kernel(x, edge_index, Wl1, bl1, Wr1, Wl2, bl2, Wr2, Wh, bh):
    src = edge_index[0].astype(jnp.int32)
    dst = edge_index[1].astype(jnp.int32)
    npad = EPAD - N_EDGES
    srcm = jnp.concatenate(
        [src, jnp.zeros((npad,), jnp.int32)]).reshape(EROWS, CH)
    dstm = jnp.concatenate(
        [dst, jnp.full((npad,), NPAD - 8, jnp.int32)]).reshape(EROWS, CH)
    z32 = jnp.zeros((NPAD, HID), _f32)
    z16 = jnp.zeros((NPAD, CW), _f32)
    ones_h = jnp.ones((CH, CW), _f32)

    p1, r1 = _pre(x, Wl1, Wr1, bl1.reshape(1, HID))
    acc1, cnt1 = _make_edge_pass(True)(p1, srcm, dstm, z32, z16, ones_h)
    p2, r2, inv = _mid(acc1[0], acc1[1], cnt1[0], cnt1[1], r1,
                       Wl2, Wr2, bl2.reshape(1, HID))
    (acc2,) = _make_edge_pass(False)(p2, srcm, dstm, z32)
    z = _post(acc2[0], acc2[1], inv, r2, Wh, bh.reshape(1, OUT_DIM))
    return z


# speedup vs baseline: 17.0578x; 1.7172x over previous
import functools

import jax
import jax.numpy as jnp
from jax import lax
from jax.experimental import pallas as pl
from jax.experimental.pallas import tpu as pltpu
from jax.experimental.pallas import tpu_sc as plsc

N_NODES = 10000
N_EDGES = 320000
IN_CH = 128
HID = 32
OUT_DIM = HID // 2
CW = 16

NW = 32
CH = 128
K = 8
RPW = 80
G = RPW // K
EROWS = NW * RPW
EPAD = EROWS * CH
NPAD = 10112
RPS = NPAD // 16


def _edge_pass_body(with_counts, *refs):
    if with_counts:
        (table, srcm, dstm, z32, z16, ones_h,
         acc_out, cnt_out, table_sh, acc_sh, cnt_sh, ones_v,
         src_v, dst_v, rows_v, i_sem, g_sem, s_sem_a, s_sem_b) = refs
    else:
        (table, srcm, dstm, z32,
         acc_out, table_sh, acc_sh,
         src_v, dst_v, rows_v, i_sem, g_sem, s_sem_a, s_sem_b) = refs

    c = lax.axis_index("c")
    s = lax.axis_index("s")
    wid = s * 2 + c

    sl = pl.ds(s * RPS, RPS)
    pltpu.sync_copy(table.at[sl], table_sh.at[sl])
    pltpu.sync_copy(z32.at[sl], acc_sh.at[sl])
    if with_counts:
        pltpu.sync_copy(z16.at[sl], cnt_sh.at[sl])
        pltpu.sync_copy(ones_h, ones_v)
    plsc.subcore_barrier()

    def idx_fetch(g, p):
        r0 = wid * RPW + g * K
        pltpu.async_copy(srcm.at[pl.ds(r0, K)], src_v.at[p], i_sem)
        pltpu.async_copy(dstm.at[pl.ds(r0, K)], dst_v.at[p], i_sem)

    def idx_wait(p):
        pltpu.make_async_copy(srcm.at[pl.ds(0, K)], src_v.at[p], i_sem).wait()
        pltpu.make_async_copy(dstm.at[pl.ds(0, K)], dst_v.at[p], i_sem).wait()

    def scatter_sem(p):
        return s_sem_a if p == 0 else s_sem_b

    def drain_scatters(p):
        sem = scatter_sem(p)
        for j in range(K):
            pltpu.make_async_copy(
                rows_v.at[0, j], acc_sh.at[pl.ds(0, CH)], sem).wait()
            if with_counts:
                pltpu.make_async_copy(
                    ones_v, cnt_sh.at[pl.ds(0, CH)], sem).wait()

    def run_chunk(g, p, drain_pred, prefetch_pred):
        idx_wait(p)
        gd = [
            pltpu.async_copy(
                table_sh.at[src_v.at[p, j]], rows_v.at[p, j], g_sem)
            for j in range(K)
        ]

        if drain_pred is True:
            drain_scatters(1 - p)
        else:
            @pl.when(drain_pred)
            def _():
                drain_scatters(1 - p)

        @pl.when(prefetch_pred)
        def _():
            idx_fetch(g + 1, 1 - p)

        for d in gd:
            d.wait()
        sem = scatter_sem(p)
        for j in range(K):
            pltpu.async_copy(
                rows_v.at[p, j], acc_sh.at[dst_v.at[p, j]], sem, add=True)
            if with_counts:
                pltpu.async_copy(
                    ones_v, cnt_sh.at[dst_v.at[p, j]], sem, add=True)

    idx_fetch(0, 0)

    def super_chunk_pair(h, carry):
        g0 = 2 * h
        run_chunk(g0, 0, h >= 1, g0 + 1 < G)
        run_chunk(g0 + 1, 1, True, g0 + 2 < G)
        return carry

    lax.fori_loop(0, G // 2, super_chunk_pair, 0)
    drain_scatters(1)
    plsc.subcore_barrier()

    pltpu.sync_copy(acc_sh.at[sl], acc_out.at[c, sl])
    if with_counts:
        pltpu.sync_copy(cnt_sh.at[sl], cnt_out.at[c, sl])


@functools.lru_cache(maxsize=None)
def _make_edge_pass(with_counts):
    f32, i32 = jnp.float32, jnp.int32
    outs = [jax.ShapeDtypeStruct((2, NPAD, HID), f32)]
    scratch = [pltpu.VMEM_SHARED((NPAD, HID), f32),
               pltpu.VMEM_SHARED((NPAD, HID), f32)]
    if with_counts:
        outs.append(jax.ShapeDtypeStruct((2, NPAD, CW), f32))
        scratch += [pltpu.VMEM_SHARED((NPAD, CW), f32), pltpu.VMEM((CH, CW), f32)]
    scratch += [
        pltpu.VMEM((2, K, CH), i32),
        pltpu.VMEM((2, K, CH), i32),
        pltpu.VMEM((2, K, CH, HID), f32),
        pltpu.SemaphoreType.DMA,
        pltpu.SemaphoreType.DMA,
        pltpu.SemaphoreType.DMA,
        pltpu.SemaphoreType.DMA,
    ]
    mesh = plsc.VectorSubcoreMesh(core_axis_name="c", subcore_axis_name="s")
    return pl.kernel(
        functools.partial(_edge_pass_body, with_counts),
        out_type=tuple(outs),
        mesh=mesh,
        scratch_types=scratch,
        compiler_params=pltpu.CompilerParams(use_tc_tiling_on_sc=False),
        name=f"sage_edge_pass_{'cnt' if with_counts else 'nocnt'}",
    )


def _dot_t(a, w):
    return lax.dot_general(a, w, (((1,), (1,)), ((), ())),
                           preferred_element_type=jnp.float32)


def _pad_rows(v):
    return jnp.concatenate(
        [v, jnp.zeros((NPAD - N_NODES, v.shape[1]), v.dtype)], axis=0)


def _pre_body(x, wl, wr, bl, p_out, r_out):
    xv = x[...]
    p_out[...] = _pad_rows(_dot_t(xv, wl[...]))
    r_out[...] = _dot_t(xv, wr[...]) + bl[...]


def _mid_body(accA, accB, cntA, cntB, rpb, wl2, wr2, bl2, p_out, r_out, inv_out):
    agg = accA[...][:N_NODES] + accB[...][:N_NODES]
    cnt = cntA[...][:N_NODES, 0:1] + cntB[...][:N_NODES, 0:1]
    inv = 1.0 / jnp.maximum(cnt, 1.0)
    h1 = jnp.maximum(agg * inv + rpb[...], 0.0)
    p_out[...] = _pad_rows(_dot_t(h1, wl2[...]))
    r_out[...] = _dot_t(h1, wr2[...]) + bl2[...]
    inv_out[...] = inv


def _post_body(accA, accB, inv, rpb, wh, bh, z_out):
    agg = accA[...][:N_NODES] + accB[...][:N_NODES]
    h2 = jnp.maximum(agg * inv[...] + rpb[...], 0.0)
    z_out[...] = _dot_t(h2, wh[...]) + bh[...]


_f32 = jnp.float32

_pre = pl.pallas_call(
    _pre_body,
    out_shape=(jax.ShapeDtypeStruct((NPAD, HID), _f32),
               jax.ShapeDtypeStruct((N_NODES, HID), _f32)),
)

_mid = pl.pallas_call(
    _mid_body,
    out_shape=(jax.ShapeDtypeStruct((NPAD, HID), _f32),
               jax.ShapeDtypeStruct((N_NODES, HID), _f32),
               jax.ShapeDtypeStruct((N_NODES, 1), _f32)),
)

_post = pl.pallas_call(
    _post_body,
    out_shape=jax.ShapeDtypeStruct((N_NODES, OUT_DIM), _f32),
)


def kernel(x, edge_index, Wl1, bl1, Wr1, Wl2, bl2, Wr2, Wh, bh):
    src = edge_index[0].astype(jnp.int32)
    dst = edge_index[1].astype(jnp.int32)
    npad = EPAD - N_EDGES
    srcm = jnp.concatenate(
        [src, jnp.zeros((npad,), jnp.int32)]).reshape(EROWS, CH)
    dstm = jnp.concatenate(
        [dst, jnp.full((npad,), NPAD - 8, jnp.int32)]).reshape(EROWS, CH)
    z32 = jnp.zeros((NPAD, HID), _f32)
    z16 = jnp.zeros((NPAD, CW), _f32)
    ones_h = jnp.ones((CH, CW), _f32)

    p1, r1 = _pre(x, Wl1, Wr1, bl1.reshape(1, HID))
    acc1, cnt1 = _make_edge_pass(True)(p1, srcm, dstm, z32, z16, ones_h)
    p2, r2, inv = _mid(acc1[0], acc1[1], cnt1[0], cnt1[1], r1,
                       Wl2, Wr2, bl2.reshape(1, HID))
    (acc2,) = _make_edge_pass(False)(p2, srcm, dstm, z32)
    z = _post(acc2[0], acc2[1], inv, r2, Wh, bh.reshape(1, OUT_DIM))
    return z
